# Initial kernel scaffold; baseline (speedup 1.0000x reference)
import jax
import jax.numpy as jnp
from jax.experimental import pallas as pl


def kernel(x, edge_index, batch, W1_l, b1_l, W1_r, W2_l, b2_l, W2_r, Wm, bm):
    raise NotImplementedError("write your pallas kernel here")


# speedup vs baseline: 3.8508x; 3.8508x over previous
import functools

import jax
import jax.numpy as jnp
from jax import lax
from jax.experimental import pallas as pl
from jax.experimental.pallas import tpu as pltpu
from jax.experimental.pallas import tpu_sc as plsc

N = 10000
D = 128
DH = D // 2
E = 320000
NC = 2
NS = 16
L = 16
CHUNK = 128
CPW = 158
HCPW = CPW // 2
E_PAD = NS * CPW * CHUNK
N_PAD = 10240
ROWS_PT = N_PAD // NS


def _agg_body(with_counts, *refs):
    if with_counts:
        (src2_hbm, dst_hbm, x2_hbm, out_hbm, cnt_hbm,
         sidx, didx, rows, zbuf, ones, zsmall, accum, cnt_acc, sem) = refs
    else:
        (src2_hbm, dst_hbm, x2_hbm, out_hbm,
         sidx, didx, rows, zbuf, accum, sem) = refs
    c = lax.axis_index("c")
    s = lax.axis_index("s")

    def zrow(i, _):
        for j in range(DH // L):
            zbuf[i, pl.ds(j * L, L)] = jnp.zeros((L,), jnp.float32)
        return 0
    lax.fori_loop(0, CHUNK, zrow, 0)
    if with_counts:
        def orow(i, _):
            ones[i, :] = jnp.ones((L,), jnp.float32)
            zsmall[i, :] = jnp.zeros((L,), jnp.float32)
            return 0
        lax.fori_loop(0, CHUNK, orow, 0)

    tb = s * ROWS_PT
    def zacc(i, _):
        pltpu.sync_copy(zbuf, accum.at[pl.ds(tb + i * CHUNK, CHUNK)])
        if with_counts:
            pltpu.sync_copy(zsmall, cnt_acc.at[pl.ds(tb + i * CHUNK, CHUNK)])
        return 0
    lax.fori_loop(0, ROWS_PT // CHUNK, zacc, 0)
    plsc.subcore_barrier()

    wbase = s * (CPW * CHUNK)
    cstart = c * HCPW
    def step(i, _):
        base = wbase + i * CHUNK
        pltpu.sync_copy(src2_hbm.at[c, pl.ds(base, CHUNK)], sidx)
        pltpu.sync_copy(dst_hbm.at[pl.ds(base, CHUNK)], didx)
        pltpu.async_copy(x2_hbm.at[sidx], rows, sem).wait()
        pltpu.sync_copy(rows, accum.at[didx], add=True)
        if with_counts:
            @pl.when((i >= cstart) & (i < cstart + HCPW))
            def _():
                pltpu.sync_copy(ones, cnt_acc.at[didx], add=True)
        return 0
    lax.fori_loop(0, CPW, step, 0)
    plsc.subcore_barrier()

    pltpu.sync_copy(accum.at[pl.ds(tb, ROWS_PT)], out_hbm.at[c, pl.ds(tb, ROWS_PT)])
    if with_counts:
        pltpu.sync_copy(cnt_acc.at[pl.ds(tb, ROWS_PT)],
                        cnt_hbm.at[c, pl.ds(tb, ROWS_PT)])


def _make_agg(with_counts):
    out_type = [jax.ShapeDtypeStruct((NC, N_PAD, DH), jnp.float32)]
    scratch = [
        pltpu.VMEM((CHUNK,), jnp.int32),
        pltpu.VMEM((CHUNK,), jnp.int32),
        pltpu.VMEM((CHUNK, DH), jnp.float32),
        pltpu.VMEM((CHUNK, DH), jnp.float32),
    ]
    if with_counts:
        out_type.append(jax.ShapeDtypeStruct((NC, N_PAD, L), jnp.float32))
        scratch.append(pltpu.VMEM((CHUNK, L), jnp.float32))
        scratch.append(pltpu.VMEM((CHUNK, L), jnp.float32))
    scratch.append(pltpu.VMEM_SHARED((N_PAD, DH), jnp.float32))
    if with_counts:
        scratch.append(pltpu.VMEM_SHARED((N_PAD, L), jnp.float32))
    scratch.append(pltpu.SemaphoreType.DMA)
    return pl.kernel(
        functools.partial(_agg_body, with_counts),
        mesh=plsc.VectorSubcoreMesh(core_axis_name="c", subcore_axis_name="s"),
        out_type=out_type,
        scratch_types=scratch,
        compiler_params=pltpu.CompilerParams(use_tc_tiling_on_sc=False),
    )


_agg_wc = _make_agg(True)
_agg_nc = _make_agg(False)


def _dotT(a, w):
    return lax.dot_general(a, w, (((1,), (1,)), ((), ())),
                           preferred_element_type=jnp.float32)


def _sage(plo, phi, c0, c1, xin, wl, bl, wr):
    cnt = jnp.maximum(c0[:, 0:1] + c1[:, 0:1], 1.0)
    r = 1.0 / cnt
    h = _dotT(plo[...] * r, wl[:, :DH]) + _dotT(phi[...] * r, wl[:, DH:])
    return h + bl[...] + _dotT(xin[...], wr[...])


def _fuse1_body(x, plo, phi, c0, c1, wl, bl, wr, o):
    o[...] = jnp.maximum(_sage(plo, phi, c0[...], c1[...], x, wl[...], bl, wr), 0.0)


def _fuse2_body(y1, plo, phi, c0, c1, wl, bl, wr, wm, bm, o):
    y2 = jnp.maximum(_sage(plo, phi, c0[...], c1[...], y1, wl[...], bl, wr), 0.0)
    o[...] = _dotT(y2, wm[...]) + bm[...]


_fuse1 = pl.pallas_call(
    _fuse1_body, out_shape=jax.ShapeDtypeStruct((N, D), jnp.float32))
_fuse2 = pl.pallas_call(
    _fuse2_body, out_shape=jax.ShapeDtypeStruct((N, D), jnp.float32))


def kernel(x, edge_index, batch, W1_l, b1_l, W1_r, W2_l, b2_l, W2_r, Wm, bm):
    ei = edge_index.astype(jnp.int32)
    pad = E_PAD - E
    src = jnp.concatenate([ei[0], jnp.zeros((pad,), jnp.int32)])
    dst = jnp.concatenate([ei[1], jnp.full((pad,), N, jnp.int32)])
    src2 = jnp.stack([2 * src, 2 * src + 1])

    p, cnt = _agg_wc(src2, dst, x.reshape(2 * N, DH))
    c0, c1 = cnt[0, :N], cnt[1, :N]
    y1 = _fuse1(x, p[0, :N], p[1, :N], c0, c1,
                W1_l, b1_l.reshape(1, D), W1_r)
    q = _agg_nc(src2, dst, y1.reshape(2 * N, DH))[0]
    return _fuse2(y1, q[0, :N], q[1, :N], c0, c1,
                  W2_l, b2_l.reshape(1, D), W2_r, Wm, bm.reshape(1, D))
